# baseline (speedup 1.0000x reference)
import functools

import jax
import jax.numpy as jnp
from jax import lax
from jax.experimental import pallas as pl
from jax.experimental.pallas import tpu as pltpu
from jax.experimental.pallas import tpu_sc as plsc

_VOCAB = 100000
_EMBED = 128
_BATCH = 1024
_SEQ = 200
_B = _BATCH * _SEQ
_NC = 2
_NS = 16
_NW = _NC * _NS
_RPW = _BATCH // _NW
_BPW = _RPW * _SEQ
_CH = 80
_NCHUNK = _BPW // _CH
_NBUF = 8
_NITER = _NCHUNK // _NBUF
_DETILE_OFFS = tuple(range(0, 192, 16)) + (184,)

_mesh = plsc.VectorSubcoreMesh(
    core_axis_name="c", subcore_axis_name="s", num_cores=_NC, num_subcores=_NS
)


@functools.partial(
    pl.kernel,
    out_type=jax.ShapeDtypeStruct((_B, _EMBED), jnp.float32),
    mesh=_mesh,
    scratch_types=[
        pltpu.VMEM((_RPW, _SEQ), jnp.int32),
        pltpu.VMEM((_BPW,), jnp.int32),
        pltpu.VMEM((_NBUF, _CH, _EMBED), jnp.float32),
        [pltpu.SemaphoreType.DMA] * _NBUF,
        [pltpu.SemaphoreType.DMA] * _NBUF,
    ],
)
def _gather_rows(idx_hbm, table_hbm, out_hbm, idx2d, idx_v, rows_v, gsems, osems):
    wid = lax.axis_index("s") * _NC + lax.axis_index("c")
    row0 = wid * _RPW
    base = row0 * _SEQ
    pltpu.sync_copy(idx_hbm.at[pl.ds(row0, _RPW)], idx2d)

    def detile_row(r, c):
        for o in _DETILE_OFFS:
            idx_v[pl.ds(r * _SEQ + o, 16)] = idx2d[r, pl.ds(o, 16)]
        return c

    lax.fori_loop(0, _RPW, detile_row, 0)

    def gather_start(g, b):
        return pltpu.async_copy(
            table_hbm.at[idx_v.at[pl.ds(g * _CH, _CH)]], rows_v.at[b], gsems[b]
        )

    def out_start(g, b):
        return pltpu.async_copy(
            rows_v.at[b], out_hbm.at[pl.ds(base + g * _CH, _CH)], osems[b]
        )

    def out_drain(b):
        pltpu.make_async_copy(
            rows_v.at[b], out_hbm.at[pl.ds(base, _CH)], osems[b]
        ).wait()

    def ring(j, first):
        gbase = j * _NBUF
        descs = []
        for b in range(_NBUF):
            if not first:
                out_drain(b)
            descs.append(gather_start(gbase + b, b))
        for b in range(_NBUF):
            descs[b].wait()
            out_start(gbase + b, b)

    ring(0, True)
    lax.fori_loop(1, _NITER, lambda j, c: (ring(j, False), c)[1], 0)
    for b in range(_NBUF):
        out_drain(b)


def kernel(prompt_ids, embedding_table):
    if prompt_ids.dtype != jnp.int32:
        prompt_ids = prompt_ids.astype(jnp.int32)
    out = _gather_rows(prompt_ids, embedding_table)
    return out.reshape(_BATCH, _SEQ, _EMBED)
